# Initial kernel scaffold; baseline (speedup 1.0000x reference)
import jax
import jax.numpy as jnp
from jax.experimental import pallas as pl


def kernel(features, conv_w, conv_b, cls_w, cls_b, reg_w, reg_b, image_shape):
    raise NotImplementedError("write your pallas kernel here")


# speedup vs baseline: 14.9305x; 14.9305x over previous
import functools
import math

import jax
import jax.numpy as jnp
import numpy as np
from jax.experimental import pallas as pl
from jax.experimental.pallas import tpu as pltpu

C_IN = 256
H = 50
W = 50
NUM_ANCHORS = 9
STRIDE = 16
PRE_NMS_TOP_N = 6000
POST_NMS_TOP_N = 1000
NMS_THRESH = 0.7
MIN_SIZE = 1.0
BBOX_XFORM_CLIP = math.log(1000.0 / 16)

GRID = 52
NFLAT = GRID * GRID
NPAD = 2816
XEXT = 2944
NSEL = 25600
SROWS = 200
NEG_INF = float("-inf")
BIG_I32 = np.int32(1 << 30)


def _build_consts():
    sizes = np.array([128.0, 256.0, 512.0])
    ratios = np.array([0.5, 1.0, 2.0])
    hs, ws = [], []
    for s in sizes:
        for r in ratios:
            hs.append(s * np.sqrt(r))
            ws.append(s / np.sqrt(r))
    hs = np.array(hs, np.float64)
    ws = np.array(ws, np.float64)

    hh = np.arange(GRID)[:, None].repeat(GRID, 1)
    ww = np.arange(GRID)[None, :].repeat(GRID, 0)
    valid = (hh >= 1) & (hh <= H) & (ww >= 1) & (ww <= W)
    h = hh - 1
    w = ww - 1
    cx = (w + 0.5) * STRIDE
    cy = (h + 0.5) * STRIDE

    def flat_pad(a2d, fill):
        flat = a2d.reshape(-1)
        out = np.full((NPAD,), fill, a2d.dtype)
        out[:NFLAT] = flat
        return out

    cxf = flat_pad(cx.astype(np.float32), 0.0)
    cyf = flat_pad(cy.astype(np.float32), 0.0)
    validf = flat_pad(valid, False)

    CW = np.broadcast_to(ws.astype(np.float32)[:, None], (NUM_ANCHORS, NPAD)).copy()
    CH = np.broadcast_to(hs.astype(np.float32)[:, None], (NUM_ANCHORS, NPAD)).copy()
    CX = np.broadcast_to(cxf[None, :], (NUM_ANCHORS, NPAD)).copy()
    CY = np.broadcast_to(cyf[None, :], (NUM_ANCHORS, NPAD)).copy()

    posmask = np.where(validf[None, :], 0.0, NEG_INF).astype(np.float32)
    posmask = np.broadcast_to(posmask, (NUM_ANCHORS, NPAD)).copy()

    hwf = flat_pad((np.minimum(h, H - 1) * W + np.minimum(w, W - 1)).astype(np.int64), 0)
    refidx = hwf[None, :] * NUM_ANCHORS + np.arange(NUM_ANCHORS)[:, None]
    refidx = np.where(np.broadcast_to(validf[None, :], refidx.shape), refidx, BIG_I32)
    refidx = refidx.astype(np.int32)
    return CW, CH, CX, CY, posmask, refidx


_CW, _CH, _CX, _CY, _POSMASK, _REFIDX = _build_consts()
_TAP_OFFS = tuple(kh * GRID + kw for kh in range(3) for kw in range(3))


def _dense_body(x_ext_ref, wconv_ref, bconv_ref, whead_ref, bhead_ref,
                cw_ref, ch_ref, cx_ref, cy_ref, posmask_ref, img_ref,
                sraw_ref, snms_ref, x1_ref, y1_ref, x2_ref, y2_ref):
    acc = jnp.zeros((C_IN, NPAD), jnp.float32)
    for k, off in enumerate(_TAP_OFFS):
        acc += jnp.dot(wconv_ref[k], x_ext_ref[:, off:off + NPAD],
                       preferred_element_type=jnp.float32)
    act = jnp.maximum(acc + bconv_ref[:, 0:1], 0.0)
    heads = jnp.dot(whead_ref[...], act, preferred_element_type=jnp.float32)
    heads = heads + bhead_ref[:, 0:1]

    logits = heads[0:9]
    dx = heads[9:18]
    dy = heads[18:27]
    dw = jnp.minimum(heads[27:36], BBOX_XFORM_CLIP)
    dh = jnp.minimum(heads[36:45], BBOX_XFORM_CLIP)

    cw = cw_ref[...]
    ch = ch_ref[...]
    pcx = dx * cw + cx_ref[...]
    pcy = dy * ch + cy_ref[...]
    pw = jnp.exp(dw) * cw
    ph = jnp.exp(dh) * ch

    img = img_ref[0, 0]
    x1 = jnp.clip(pcx - 0.5 * pw, 0.0, img)
    y1 = jnp.clip(pcy - 0.5 * ph, 0.0, img)
    x2 = jnp.clip(pcx + 0.5 * pw, 0.0, img)
    y2 = jnp.clip(pcy + 0.5 * ph, 0.0, img)

    sraw = logits + posmask_ref[...]
    valid = ((x2 - x1) >= MIN_SIZE) & ((y2 - y1) >= MIN_SIZE)
    snms = jnp.where(valid, sraw, NEG_INF)

    sraw_ref[...] = sraw
    snms_ref[...] = snms
    x1_ref[...] = x1
    y1_ref[...] = y1
    x2_ref[...] = x2
    y2_ref[...] = y2


def _count_ge(keys, cand):
    return jnp.sum((keys >= cand).astype(jnp.int32))


def _nms_body(sraw_ref, snms_ref, x1_ref, y1_ref, x2_ref, y2_ref,
              refidx_ref, myiota_ref, ehot_ref, out_ref):
    sraw = sraw_ref[...]
    bits = jax.lax.bitcast_convert_type(sraw, jnp.int32)
    keys = jnp.where(bits < 0, bits ^ jnp.int32(0x7FFFFFFF), bits)

    cpos = _count_ge(keys, jnp.int32(0))
    k_val = jnp.where(cpos >= PRE_NMS_TOP_N, jnp.int32(0), jnp.int32(-2147483648))
    for bit in range(30, -1, -1):
        cand = k_val | jnp.int32(1 << bit)
        k_val = jnp.where(_count_ge(keys, cand) >= PRE_NMS_TOP_N, cand, k_val)

    c_gt = jnp.sum((keys > k_val).astype(jnp.int32))
    m_ties = PRE_NMS_TOP_N - c_gt
    tie = keys == k_val
    refidx = refidx_ref[...]
    lo = jnp.int32(0)
    hi = jnp.int32((1 << 15) - 1)
    for _ in range(15):
        mid = (lo + hi) // 2
        cnt = jnp.sum((tie & (refidx <= mid)).astype(jnp.int32))
        take = cnt >= m_ties
        hi = jnp.where(take, mid, hi)
        lo = jnp.where(take, lo, mid + 1)
    in_topk = (keys > k_val) | (tie & (refidx <= hi))

    s0 = jnp.where(in_topk, snms_ref[...], NEG_INF)

    x1 = x1_ref[...]
    y1 = y1_ref[...]
    x2 = x2_ref[...]
    y2 = y2_ref[...]
    areas = (x2 - x1) * (y2 - y1)
    myiota = myiota_ref[...]

    m0 = jnp.max(sraw)
    i0 = jnp.min(jnp.where(sraw == m0, refidx, BIG_I32))
    ch0 = ((sraw == m0) & (refidx == i0)).astype(jnp.float32)
    fb0 = jnp.sum(ch0 * x1)
    fb1 = jnp.sum(ch0 * y1)
    fb2 = jnp.sum(ch0 * x2)
    fb3 = jnp.sum(ch0 * y2)
    fba = jnp.sum(ch0 * areas)

    e0 = ehot_ref[0:1, :]
    e1 = ehot_ref[1:2, :]
    e2 = ehot_ref[2:3, :]
    e3 = ehot_ref[3:4, :]

    def body(it, s):
        mval = jnp.max(s)
        sel = s == mval
        ii = jnp.min(jnp.where(sel, myiota, BIG_I32))
        chosen = sel & (myiota == ii)
        cf = chosen.astype(jnp.float32)
        isfb = mval == NEG_INF
        b0 = jnp.where(isfb, fb0, jnp.sum(cf * x1))
        b1 = jnp.where(isfb, fb1, jnp.sum(cf * y1))
        b2 = jnp.where(isfb, fb2, jnp.sum(cf * x2))
        b3 = jnp.where(isfb, fb3, jnp.sum(cf * y2))
        ar = jnp.where(isfb, fba, jnp.sum(cf * areas))

        xx1 = jnp.maximum(b0, x1)
        yy1 = jnp.maximum(b1, y1)
        xx2 = jnp.minimum(b2, x2)
        yy2 = jnp.minimum(b3, y2)
        inter = jnp.maximum(xx2 - xx1, 0.0) * jnp.maximum(yy2 - yy1, 0.0)
        iou = inter / (ar + areas - inter + 1e-9)
        s = jnp.where(iou > NMS_THRESH, NEG_INF, s)
        s = jnp.where(chosen, NEG_INF, s)

        out_ref[pl.ds(it, 1), :] = b0 * e0 + b1 * e1 + b2 * e2 + b3 * e3
        return s

    jax.lax.fori_loop(0, POST_NMS_TOP_N, body, s0)


@functools.partial(jax.jit, static_argnames=())
def kernel(features, conv_w, conv_b, cls_w, cls_b, reg_w, reg_b, image_shape):
    f32 = jnp.float32

    x = features[0].astype(f32)
    x_pad = jnp.pad(x, ((0, 0), (1, 1), (1, 1)))
    x_flat = x_pad.reshape(C_IN, NFLAT)
    x_ext = jnp.pad(x_flat, ((0, 0), (53, XEXT - NFLAT - 53)))

    wconv = jnp.transpose(conv_w, (2, 3, 0, 1)).reshape(9, C_IN, C_IN)
    bconv = conv_b.reshape(C_IN, 1)

    wcls = cls_w.reshape(NUM_ANCHORS, C_IN)
    wreg = reg_w.reshape(NUM_ANCHORS, 4, C_IN).transpose(1, 0, 2).reshape(36, C_IN)
    whead = jnp.concatenate([wcls, wreg, jnp.zeros((3, C_IN), f32)], axis=0)
    bcls = cls_b.reshape(NUM_ANCHORS)
    breg = reg_b.reshape(NUM_ANCHORS, 4).transpose(1, 0).reshape(36)
    bhead = jnp.concatenate([bcls, breg, jnp.zeros((3,), f32)]).reshape(48, 1)

    img = jnp.asarray(image_shape, f32).reshape(1, 1)

    shp = (NUM_ANCHORS, NPAD)
    dense_out = pl.pallas_call(
        _dense_body,
        out_shape=[jax.ShapeDtypeStruct(shp, f32)] * 6,
        in_specs=[
            pl.BlockSpec(memory_space=pltpu.VMEM),
            pl.BlockSpec(memory_space=pltpu.VMEM),
            pl.BlockSpec(memory_space=pltpu.VMEM),
            pl.BlockSpec(memory_space=pltpu.VMEM),
            pl.BlockSpec(memory_space=pltpu.VMEM),
            pl.BlockSpec(memory_space=pltpu.VMEM),
            pl.BlockSpec(memory_space=pltpu.VMEM),
            pl.BlockSpec(memory_space=pltpu.VMEM),
            pl.BlockSpec(memory_space=pltpu.VMEM),
            pl.BlockSpec(memory_space=pltpu.VMEM),
            pl.BlockSpec(memory_space=pltpu.SMEM),
        ],
        out_specs=[pl.BlockSpec(memory_space=pltpu.VMEM)] * 6,
    )(x_ext, wconv, bconv, whead, bhead,
      jnp.asarray(_CW), jnp.asarray(_CH), jnp.asarray(_CX), jnp.asarray(_CY),
      jnp.asarray(_POSMASK), img)

    sraw, snms, x1, y1, x2, y2 = dense_out

    def to_sel(a, fill):
        flat = a.reshape(-1)
        return jnp.pad(flat, (0, NSEL - flat.shape[0]),
                       constant_values=fill).reshape(SROWS, 128)

    sraw_s = to_sel(sraw, NEG_INF)
    snms_s = to_sel(snms, NEG_INF)
    x1_s = to_sel(x1, 0.0)
    y1_s = to_sel(y1, 0.0)
    x2_s = to_sel(x2, 0.0)
    y2_s = to_sel(y2, 0.0)

    refidx_s = jnp.pad(jnp.asarray(_REFIDX).reshape(-1),
                       (0, NSEL - NUM_ANCHORS * NPAD),
                       constant_values=BIG_I32).reshape(SROWS, 128)
    myiota_s = jnp.arange(NSEL, dtype=jnp.int32).reshape(SROWS, 128)
    ehot = jnp.zeros((8, 128), f32).at[jnp.arange(4), jnp.arange(4)].set(1.0)

    out = pl.pallas_call(
        _nms_body,
        out_shape=jax.ShapeDtypeStruct((1024, 128), f32),
        in_specs=[pl.BlockSpec(memory_space=pltpu.VMEM)] * 9,
        out_specs=pl.BlockSpec(memory_space=pltpu.VMEM),
    )(sraw_s, snms_s, x1_s, y1_s, x2_s, y2_s, refidx_s, myiota_s, ehot)

    return out[:POST_NMS_TOP_N, :4]


# speedup vs baseline: 24.3406x; 1.6303x over previous
import functools
import math

import jax
import jax.numpy as jnp
import numpy as np
from jax.experimental import pallas as pl
from jax.experimental.pallas import tpu as pltpu

C_IN = 256
H = 50
W = 50
NUM_ANCHORS = 9
STRIDE = 16
PRE_NMS_TOP_N = 6000
POST_NMS_TOP_N = 1000
NMS_THRESH = 0.7
MIN_SIZE = 1.0
BBOX_XFORM_CLIP = math.log(1000.0 / 16)

GRID = 52
NFLAT = GRID * GRID
NPAD = 2816
XEXT = 2944
NSEL = 25600
SROWS = 200
NEG_INF = float("-inf")
BIG_I32 = np.int32(1 << 30)


def _build_consts():
    sizes = np.array([128.0, 256.0, 512.0])
    ratios = np.array([0.5, 1.0, 2.0])
    hs, ws = [], []
    for s in sizes:
        for r in ratios:
            hs.append(s * np.sqrt(r))
            ws.append(s / np.sqrt(r))
    hs = np.array(hs, np.float64)
    ws = np.array(ws, np.float64)

    hh = np.arange(GRID)[:, None].repeat(GRID, 1)
    ww = np.arange(GRID)[None, :].repeat(GRID, 0)
    valid = (hh >= 1) & (hh <= H) & (ww >= 1) & (ww <= W)
    h = hh - 1
    w = ww - 1
    cx = (w + 0.5) * STRIDE
    cy = (h + 0.5) * STRIDE

    def flat_pad(a2d, fill):
        flat = a2d.reshape(-1)
        out = np.full((NPAD,), fill, a2d.dtype)
        out[:NFLAT] = flat
        return out

    cxf = flat_pad(cx.astype(np.float32), 0.0)
    cyf = flat_pad(cy.astype(np.float32), 0.0)
    validf = flat_pad(valid, False)

    CW = np.broadcast_to(ws.astype(np.float32)[:, None], (NUM_ANCHORS, NPAD)).copy()
    CH = np.broadcast_to(hs.astype(np.float32)[:, None], (NUM_ANCHORS, NPAD)).copy()
    CX = np.broadcast_to(cxf[None, :], (NUM_ANCHORS, NPAD)).copy()
    CY = np.broadcast_to(cyf[None, :], (NUM_ANCHORS, NPAD)).copy()

    posmask = np.where(validf[None, :], 0.0, NEG_INF).astype(np.float32)
    posmask = np.broadcast_to(posmask, (NUM_ANCHORS, NPAD)).copy()

    hwf = flat_pad((np.minimum(h, H - 1) * W + np.minimum(w, W - 1)).astype(np.int64), 0)
    refidx = hwf[None, :] * NUM_ANCHORS + np.arange(NUM_ANCHORS)[:, None]
    refidx = np.where(np.broadcast_to(validf[None, :], refidx.shape), refidx, BIG_I32)
    refidx = refidx.astype(np.int32)
    return CW, CH, CX, CY, posmask, refidx


_CW, _CH, _CX, _CY, _POSMASK, _REFIDX = _build_consts()
_TAP_OFFS = tuple(kh * GRID + kw for kh in range(3) for kw in range(3))


def _dense_body(x_ext_ref, wconv_ref, bconv_ref, whead_ref, bhead_ref,
                cw_ref, ch_ref, cx_ref, cy_ref, posmask_ref, img_ref,
                sraw_ref, snms_ref, x1_ref, y1_ref, x2_ref, y2_ref):
    acc = jnp.zeros((C_IN, NPAD), jnp.float32)
    for k, off in enumerate(_TAP_OFFS):
        acc += jnp.dot(wconv_ref[k], x_ext_ref[:, off:off + NPAD],
                       preferred_element_type=jnp.float32)
    act = jnp.maximum(acc + bconv_ref[:, 0:1], 0.0)
    heads = jnp.dot(whead_ref[...], act, preferred_element_type=jnp.float32)
    heads = heads + bhead_ref[:, 0:1]

    logits = heads[0:9]
    dx = heads[9:18]
    dy = heads[18:27]
    dw = jnp.minimum(heads[27:36], BBOX_XFORM_CLIP)
    dh = jnp.minimum(heads[36:45], BBOX_XFORM_CLIP)

    cw = cw_ref[...]
    ch = ch_ref[...]
    pcx = dx * cw + cx_ref[...]
    pcy = dy * ch + cy_ref[...]
    pw = jnp.exp(dw) * cw
    ph = jnp.exp(dh) * ch

    img = img_ref[0, 0]
    x1 = jnp.clip(pcx - 0.5 * pw, 0.0, img)
    y1 = jnp.clip(pcy - 0.5 * ph, 0.0, img)
    x2 = jnp.clip(pcx + 0.5 * pw, 0.0, img)
    y2 = jnp.clip(pcy + 0.5 * ph, 0.0, img)

    sraw = logits + posmask_ref[...]
    valid = ((x2 - x1) >= MIN_SIZE) & ((y2 - y1) >= MIN_SIZE)
    snms = jnp.where(valid, sraw, NEG_INF)

    sraw_ref[...] = sraw
    snms_ref[...] = snms
    x1_ref[...] = x1
    y1_ref[...] = y1
    x2_ref[...] = x2
    y2_ref[...] = y2


def _count_ge(keys, cand):
    return jnp.sum((keys >= cand).astype(jnp.int32))


def _nms_body(sraw_ref, snms_ref, x1_ref, y1_ref, x2_ref, y2_ref,
              refidx_ref, ehot_ref, out_ref):
    sraw = sraw_ref[...]
    bits = jax.lax.bitcast_convert_type(sraw, jnp.int32)
    keys = jnp.where(bits < 0, bits ^ jnp.int32(0x7FFFFFFF), bits)

    cpos = _count_ge(keys, jnp.int32(0))
    k_val = jnp.where(cpos >= PRE_NMS_TOP_N, jnp.int32(0), jnp.int32(-2147483648))
    for bit in range(30, -1, -1):
        cand = k_val | jnp.int32(1 << bit)
        k_val = jnp.where(_count_ge(keys, cand) >= PRE_NMS_TOP_N, cand, k_val)

    c_gt = jnp.sum((keys > k_val).astype(jnp.int32))
    m_ties = PRE_NMS_TOP_N - c_gt
    tie = keys == k_val
    refidx = refidx_ref[...]
    lo = jnp.int32(0)
    hi = jnp.int32((1 << 15) - 1)
    for _ in range(15):
        mid = (lo + hi) // 2
        cnt = jnp.sum((tie & (refidx <= mid)).astype(jnp.int32))
        take = cnt >= m_ties
        hi = jnp.where(take, mid, hi)
        lo = jnp.where(take, lo, mid + 1)
    in_topk = (keys > k_val) | (tie & (refidx <= hi))

    s0 = jnp.where(in_topk, snms_ref[...], NEG_INF)

    x1 = x1_ref[...]
    y1 = y1_ref[...]
    x2 = x2_ref[...]
    y2 = y2_ref[...]
    areas = (x2 - x1) * (y2 - y1)

    m0 = jnp.max(sraw)
    i0 = jnp.min(jnp.where(sraw == m0, refidx, BIG_I32))
    ch0 = ((sraw == m0) & (refidx == i0)).astype(jnp.float32)
    fb0 = jnp.sum(ch0 * x1)
    fb1 = jnp.sum(ch0 * y1)
    fb2 = jnp.sum(ch0 * x2)
    fb3 = jnp.sum(ch0 * y2)

    e0 = ehot_ref[0:1, :]
    e1 = ehot_ref[1:2, :]
    e2 = ehot_ref[2:3, :]
    e3 = ehot_ref[3:4, :]

    def body(it, s):
        mval = jnp.max(s)
        cf = (s == mval).astype(jnp.float32)
        isfb = mval == NEG_INF
        b0 = jnp.where(isfb, fb0, jnp.sum(cf * x1))
        b1 = jnp.where(isfb, fb1, jnp.sum(cf * y1))
        b2 = jnp.where(isfb, fb2, jnp.sum(cf * x2))
        b3 = jnp.where(isfb, fb3, jnp.sum(cf * y2))
        ar = (b2 - b0) * (b3 - b1)

        xx1 = jnp.maximum(b0, x1)
        yy1 = jnp.maximum(b1, y1)
        xx2 = jnp.minimum(b2, x2)
        yy2 = jnp.minimum(b3, y2)
        inter = jnp.maximum(xx2 - xx1, 0.0) * jnp.maximum(yy2 - yy1, 0.0)
        iou = inter / (ar + areas - inter + 1e-9)
        s = jnp.where(iou > NMS_THRESH, NEG_INF, s)

        out_ref[pl.ds(it, 1), :] = b0 * e0 + b1 * e1 + b2 * e2 + b3 * e3
        return s

    jax.lax.fori_loop(0, POST_NMS_TOP_N, body, s0)


@functools.partial(jax.jit, static_argnames=())
def kernel(features, conv_w, conv_b, cls_w, cls_b, reg_w, reg_b, image_shape):
    f32 = jnp.float32

    x = features[0].astype(f32)
    x_pad = jnp.pad(x, ((0, 0), (1, 1), (1, 1)))
    x_flat = x_pad.reshape(C_IN, NFLAT)
    x_ext = jnp.pad(x_flat, ((0, 0), (53, XEXT - NFLAT - 53)))

    wconv = jnp.transpose(conv_w, (2, 3, 0, 1)).reshape(9, C_IN, C_IN)
    bconv = conv_b.reshape(C_IN, 1)

    wcls = cls_w.reshape(NUM_ANCHORS, C_IN)
    wreg = reg_w.reshape(NUM_ANCHORS, 4, C_IN).transpose(1, 0, 2).reshape(36, C_IN)
    whead = jnp.concatenate([wcls, wreg, jnp.zeros((3, C_IN), f32)], axis=0)
    bcls = cls_b.reshape(NUM_ANCHORS)
    breg = reg_b.reshape(NUM_ANCHORS, 4).transpose(1, 0).reshape(36)
    bhead = jnp.concatenate([bcls, breg, jnp.zeros((3,), f32)]).reshape(48, 1)

    img = jnp.asarray(image_shape, f32).reshape(1, 1)

    shp = (NUM_ANCHORS, NPAD)
    dense_out = pl.pallas_call(
        _dense_body,
        out_shape=[jax.ShapeDtypeStruct(shp, f32)] * 6,
        in_specs=[
            pl.BlockSpec(memory_space=pltpu.VMEM),
            pl.BlockSpec(memory_space=pltpu.VMEM),
            pl.BlockSpec(memory_space=pltpu.VMEM),
            pl.BlockSpec(memory_space=pltpu.VMEM),
            pl.BlockSpec(memory_space=pltpu.VMEM),
            pl.BlockSpec(memory_space=pltpu.VMEM),
            pl.BlockSpec(memory_space=pltpu.VMEM),
            pl.BlockSpec(memory_space=pltpu.VMEM),
            pl.BlockSpec(memory_space=pltpu.VMEM),
            pl.BlockSpec(memory_space=pltpu.VMEM),
            pl.BlockSpec(memory_space=pltpu.SMEM),
        ],
        out_specs=[pl.BlockSpec(memory_space=pltpu.VMEM)] * 6,
    )(x_ext, wconv, bconv, whead, bhead,
      jnp.asarray(_CW), jnp.asarray(_CH), jnp.asarray(_CX), jnp.asarray(_CY),
      jnp.asarray(_POSMASK), img)

    sraw, snms, x1, y1, x2, y2 = dense_out

    def to_sel(a, fill):
        flat = a.reshape(-1)
        return jnp.pad(flat, (0, NSEL - flat.shape[0]),
                       constant_values=fill).reshape(SROWS, 128)

    sraw_s = to_sel(sraw, NEG_INF)
    snms_s = to_sel(snms, NEG_INF)
    x1_s = to_sel(x1, 0.0)
    y1_s = to_sel(y1, 0.0)
    x2_s = to_sel(x2, 0.0)
    y2_s = to_sel(y2, 0.0)

    refidx_s = jnp.pad(jnp.asarray(_REFIDX).reshape(-1),
                       (0, NSEL - NUM_ANCHORS * NPAD),
                       constant_values=BIG_I32).reshape(SROWS, 128)
    ehot = jnp.zeros((8, 128), f32).at[jnp.arange(4), jnp.arange(4)].set(1.0)

    out = pl.pallas_call(
        _nms_body,
        out_shape=jax.ShapeDtypeStruct((1024, 128), f32),
        in_specs=[pl.BlockSpec(memory_space=pltpu.VMEM)] * 8,
        out_specs=pl.BlockSpec(memory_space=pltpu.VMEM),
    )(sraw_s, snms_s, x1_s, y1_s, x2_s, y2_s, refidx_s, ehot)

    return out[:POST_NMS_TOP_N, :4]
